# baseline (speedup 1.0000x reference)
import functools

import jax
import jax.numpy as jnp
from jax import lax
from jax.experimental import pallas as pl
from jax.experimental.pallas import tpu as pltpu
from jax.experimental.pallas import tpu_sc as plsc

N, K, D = 16384, 64, 64
BLK = 512
GB = 512
NGB = N // GB
NW = 32
PW = N // NW


def _stage_a(xt_ref, x_ref, c_ref, lt_ref,
             pos_ref, counts_ref, sums_ref, off_ref, xp_ref,
             pred_s, posl_s, *, nblk):
    p = pl.program_id(0)
    i = pl.program_id(1)

    @pl.when((p == 0) & (i == 0))
    def _init():
        counts_ref[:, :] = jnp.zeros_like(counts_ref)
        sums_ref[:, :] = jnp.zeros_like(sums_ref)

    kio = lax.broadcasted_iota(jnp.int32, (K, BLK), 0)

    @pl.when(p == 0)
    def _phase0():
        xt = xt_ref[:, :]
        xb = x_ref[:, :]
        c = c_ref[:, :]

        cn = jnp.sum(c * c, axis=1, keepdims=True)
        xn = jnp.sum(xt * xt, axis=0, keepdims=True)
        d2 = cn - 2.0 * jnp.dot(c, xt, preferred_element_type=jnp.float32) + xn

        dmin = jnp.min(d2, axis=0, keepdims=True)
        pred = jnp.min(jnp.where(d2 <= dmin, kio, K), axis=0, keepdims=True)
        oh = (kio == pred).astype(jnp.float32)
        oh_bf = oh.astype(jnp.bfloat16)

        cum = jnp.dot(oh_bf, lt_ref[:, :], preferred_element_type=jnp.float32)
        carry = counts_ref[:, :]
        posl = jnp.sum(oh * (cum + carry), axis=0, keepdims=True)

        pred_s[pl.ds(i, 1), :] = pred
        posl_s[pl.ds(i, 1), :] = posl.astype(jnp.int32)

        counts_ref[:, :] = carry + jnp.sum(oh, axis=1, keepdims=True)
        sums_ref[:, :] += jnp.dot(oh_bf, xb.astype(jnp.bfloat16),
                                  preferred_element_type=jnp.float32)

        @pl.when(i == nblk - 1)
        def _epilogue():
            lk = (lax.broadcasted_iota(jnp.int32, (K, K), 1)
                  < lax.broadcasted_iota(jnp.int32, (K, K), 0)
                  ).astype(jnp.float32)
            off_ref[:, :] = jnp.dot(lk, counts_ref[:, :],
                                    preferred_element_type=jnp.float32)

    @pl.when(p == 1)
    def _phase1():
        predb = pred_s[pl.ds(i, 1), :]
        poslb = posl_s[pl.ds(i, 1), :]
        oh = (kio == predb).astype(jnp.float32)
        offb = jnp.sum(oh * off_ref[:, :], axis=0, keepdims=True)
        pos = poslb + offb.astype(jnp.int32)
        pos_ref[:, :, :] = pos.reshape(1, 1, BLK)
        xb = x_ref[:, :]
        xp_ref[:, :] = jnp.concatenate([xb, jnp.zeros_like(xb)], axis=1)


def _run_stage_a(x, xt, centers, lt):
    nblk = N // BLK
    return pl.pallas_call(
        functools.partial(_stage_a, nblk=nblk),
        grid=(2, nblk),
        in_specs=[
            pl.BlockSpec((D, BLK), lambda p, i: (0, i)),
            pl.BlockSpec((BLK, D), lambda p, i: (i, 0)),
            pl.BlockSpec((K, D), lambda p, i: (0, 0)),
            pl.BlockSpec((BLK, BLK), lambda p, i: (0, 0)),
        ],
        out_specs=[
            pl.BlockSpec((1, 1, BLK),
                         lambda p, i: (jnp.where(p == 0, 0, i), 0, 0)),
            pl.BlockSpec((K, 1), lambda p, i: (0, 0)),
            pl.BlockSpec((K, D), lambda p, i: (0, 0)),
            pl.BlockSpec((K, 1), lambda p, i: (0, 0)),
            pl.BlockSpec((BLK, 2 * D),
                         lambda p, i: (jnp.where(p == 1, i, 0), 0)),
        ],
        out_shape=[
            jax.ShapeDtypeStruct((nblk, 1, BLK), jnp.int32),
            jax.ShapeDtypeStruct((K, 1), jnp.float32),
            jax.ShapeDtypeStruct((K, D), jnp.float32),
            jax.ShapeDtypeStruct((K, 1), jnp.float32),
            jax.ShapeDtypeStruct((N, 2 * D), jnp.float32),
        ],
        scratch_shapes=[
            pltpu.VMEM((nblk, BLK), jnp.int32),
            pltpu.VMEM((nblk, BLK), jnp.int32),
        ],
        compiler_params=pltpu.CompilerParams(
            dimension_semantics=("arbitrary", "arbitrary"),
        ),
    )(xt, x, centers, lt)


def _scatter_body(x_hbm, pos_hbm, out_hbm, idx_v, rows_v, sem):
    wid = lax.axis_index("s") * 2 + lax.axis_index("c")
    base = wid * PW
    pltpu.sync_copy(pos_hbm.at[pl.ds(base, PW)], idx_v)
    pltpu.sync_copy(x_hbm.at[pl.ds(base, PW)], rows_v)
    pltpu.async_copy(rows_v, out_hbm.at[idx_v], sem).wait()


def _scatter_rows(xp, pos_i):
    mesh = plsc.VectorSubcoreMesh(core_axis_name="c", subcore_axis_name="s")
    fn = pl.kernel(
        _scatter_body,
        out_type=jax.ShapeDtypeStruct((N, 2 * D), jnp.float32),
        mesh=mesh,
        scratch_types=[
            pltpu.VMEM((PW,), jnp.int32),
            pltpu.VMEM((PW, 2 * D), jnp.float32),
            pltpu.SemaphoreType.DMA,
        ],
    )
    return fn(xp, pos_i)


def _gram_body(xs_ref, g_ref):
    xsb = xs_ref[:, :D].astype(jnp.bfloat16)
    g_ref[:, :] = lax.dot_general(xsb, xsb, (((0,), (0,)), ((), ())),
                                  preferred_element_type=jnp.float32)


def _run_gram(xs):
    return pl.pallas_call(
        _gram_body,
        grid=(NGB,),
        in_specs=[pl.BlockSpec((GB, 2 * D), lambda b: (b, 0))],
        out_specs=pl.BlockSpec((D, D), lambda b: (b, 0)),
        out_shape=jax.ShapeDtypeStruct((NGB * D, D), jnp.float32),
        compiler_params=pltpu.CompilerParams(
            dimension_semantics=("arbitrary",),
        ),
    )(xs)


def _stage_d(bep_ref, mlim_ref, xs_ref, g_ref, bev_ref, bsv_ref,
             counts_ref, sums_ref, ft_ref, mt_ref, ct_ref, out_ref,
             sedge, racc, prevp):
    k = pl.program_id(0)

    @pl.when(k == 0)
    def _init():
        racc[:, :] = jnp.zeros_like(racc)
        prevp[:, :] = jnp.zeros_like(prevp)

    xsb = xs_ref[:, :D]
    mlim = mlim_ref[k]
    msk = (lax.broadcasted_iota(jnp.int32, (GB, 1), 0) < mlim
           ).astype(jnp.float32)
    xs_bf = xsb.astype(jnp.bfloat16)
    xm_bf = (xsb * msk).astype(jnp.bfloat16)
    pe = lax.dot_general(xm_bf, xs_bf, (((0,), (0,)), ((), ())),
                         preferred_element_type=jnp.float32)
    sedge[pl.ds(k * D, D), :] = pe - prevp[:, :]
    prevp[:, :] = pe

    kf = lax.convert_element_type(k, jnp.float32)
    dcol = ((kf < bev_ref[:, :]).astype(jnp.float32)
            - (kf < bsv_ref[:, :]).astype(jnp.float32))
    m_dcol = jnp.reshape(jnp.broadcast_to(dcol[:, :, None], (K, D, 1)),
                         (K * D, 1))
    g_t = jnp.reshape(jnp.broadcast_to(g_ref[:, :][None, :, :], (K, D, D)),
                      (K * D, D))
    racc[:, :] += m_dcol * g_t

    @pl.when(k == K - 1)
    def _epilogue():
        s_flat = sedge[:, :] + racc[:, :]
        counts = counts_ref[:, :]
        safe = jnp.maximum(counts, 1.0)
        means = sums_ref[:, :] / safe

        filling = counts / jnp.float32(N)
        loss_fil = jnp.sum((filling - ft_ref[:, :]) ** 2,
                           axis=(0, 1), keepdims=True) / jnp.float32(K)
        loss_means = jnp.sum((means - mt_ref[:, :]) ** 2,
                             axis=(0, 1), keepdims=True) / jnp.float32(K * D)

        m3 = jnp.reshape(jnp.broadcast_to(means[:, None, :], (K, D, D)),
                         (K * D, D))
        rio = lax.broadcasted_iota(jnp.int32, (K * D, D), 0)
        jio = lax.broadcasted_iota(jnp.int32, (K * D, D), 1)
        isel = (rio % D == jio).astype(jnp.float32)
        m4 = jnp.sum(m3 * isel, axis=1, keepdims=True)

        countsb = jnp.reshape(jnp.broadcast_to(counts[:, :, None], (K, D, 1)),
                              (K * D, 1))
        denomb = jnp.maximum(countsb - 1.0, 1.0)
        covs = (s_flat - countsb * (m4 * m3)) / denomb
        loss_covs = jnp.sum((covs - ct_ref[:, :]) ** 2,
                            axis=(0, 1), keepdims=True) / jnp.float32(K * D * D)

        out_ref[:, :] = loss_fil + loss_means + loss_covs


def _run_stage_d(bep, mlim, xs, g, bev, bsv, counts, sums, ft, mt, ct):
    grid_spec = pltpu.PrefetchScalarGridSpec(
        num_scalar_prefetch=2,
        grid=(K,),
        in_specs=[
            pl.BlockSpec((GB, 2 * D), lambda k, bep, mlim: (bep[k], 0)),
            pl.BlockSpec((D, D), lambda k, bep, mlim: (jnp.minimum(k, NGB - 1), 0)),
            pl.BlockSpec((K, 1), lambda k, bep, mlim: (0, 0)),
            pl.BlockSpec((K, 1), lambda k, bep, mlim: (0, 0)),
            pl.BlockSpec((K, 1), lambda k, bep, mlim: (0, 0)),
            pl.BlockSpec((K, D), lambda k, bep, mlim: (0, 0)),
            pl.BlockSpec((K, 1), lambda k, bep, mlim: (0, 0)),
            pl.BlockSpec((K, D), lambda k, bep, mlim: (0, 0)),
            pl.BlockSpec((K * D, D), lambda k, bep, mlim: (0, 0)),
        ],
        out_specs=pl.BlockSpec((1, 1), lambda k, bep, mlim: (0, 0)),
        scratch_shapes=[
            pltpu.VMEM((K * D, D), jnp.float32),
            pltpu.VMEM((K * D, D), jnp.float32),
            pltpu.VMEM((D, D), jnp.float32),
        ],
    )
    return pl.pallas_call(
        _stage_d,
        grid_spec=grid_spec,
        out_shape=jax.ShapeDtypeStruct((1, 1), jnp.float32),
        compiler_params=pltpu.CompilerParams(
            dimension_semantics=("arbitrary",),
        ),
    )(bep, mlim, xs, g, bev, bsv, counts, sums, ft, mt, ct)


def kernel(x, cluster_centers, filling_target, means_target, covs_target):
    xt = x.T
    jio = lax.broadcasted_iota(jnp.int32, (BLK, BLK), 1)
    iio = lax.broadcasted_iota(jnp.int32, (BLK, BLK), 0)
    lt = (iio < jio).astype(jnp.bfloat16)

    pos3, counts, sums, off, xp = _run_stage_a(x, xt, cluster_centers, lt)

    pos_i = pos3.reshape(N)
    off_i = off.astype(jnp.int32).reshape(K)
    xs = _scatter_rows(xp, pos_i)

    g = _run_gram(xs)

    e_i = off_i + counts.astype(jnp.int32).reshape(K)
    bep = jnp.clip(e_i // GB, 0, NGB - 1).astype(jnp.int32)
    mlim = (e_i - bep * GB).astype(jnp.int32)
    bev = bep.astype(jnp.float32).reshape(K, 1)
    bsv = jnp.concatenate([jnp.zeros((1, 1), jnp.float32), bev[:-1]], axis=0)

    ft = filling_target.reshape(K, 1)
    ct = covs_target.reshape(K * D, D)
    out = _run_stage_d(bep, mlim, xs, g, bev, bsv, counts, sums,
                       ft, means_target, ct)
    return out[0, 0]


# speedup vs baseline: 1.0348x; 1.0348x over previous
import functools

import jax
import jax.numpy as jnp
from jax import lax
from jax.experimental import pallas as pl
from jax.experimental.pallas import tpu as pltpu
from jax.experimental.pallas import tpu_sc as plsc

N, K, D = 16384, 64, 64
BLK = 512
GB = 512
NGB = N // GB
NW = 32
PW = N // NW


def _stage_a(xt_ref, x_ref, c_ref, lt_ref,
             pos_ref, counts_ref, sums_ref, off_ref, xp_ref,
             pred_s, posl_s, *, nblk):
    p = pl.program_id(0)
    i = pl.program_id(1)

    @pl.when((p == 0) & (i == 0))
    def _init():
        counts_ref[:, :] = jnp.zeros_like(counts_ref)
        sums_ref[:, :] = jnp.zeros_like(sums_ref)

    kio = lax.broadcasted_iota(jnp.int32, (K, BLK), 0)

    @pl.when(p == 0)
    def _phase0():
        xt = xt_ref[:, :]
        xb = x_ref[:, :]
        c = c_ref[:, :]

        cn = jnp.sum(c * c, axis=1, keepdims=True)
        ones_1d = jnp.ones((1, D), jnp.float32)
        xn = jnp.dot(ones_1d, xt * xt, preferred_element_type=jnp.float32)
        d2 = cn - 2.0 * jnp.dot(c, xt, preferred_element_type=jnp.float32) + xn

        pk = (lax.bitcast_convert_type(d2, jnp.int32) & jnp.int32(-64)) | kio
        m = jnp.min(pk, axis=0, keepdims=True)
        ohb = pk == m
        oh = ohb.astype(jnp.float32)
        oh_bf = ohb.astype(jnp.bfloat16)
        pred = m & 63

        cum = jnp.dot(oh_bf, lt_ref[:, :], preferred_element_type=jnp.float32)
        carry_t = counts_ref[:, :].reshape(1, K)
        ones_1k = jnp.ones((1, K), jnp.float32)
        posl = (jnp.dot(ones_1k, oh * cum, preferred_element_type=jnp.float32)
                + jnp.dot(carry_t, oh, preferred_element_type=jnp.float32))

        pred_s[pl.ds(i, 1), :] = pred
        posl_s[pl.ds(i, 1), :] = posl.astype(jnp.int32)

        counts_ref[:, :] += jnp.dot(oh_bf, jnp.ones((BLK, 1), jnp.bfloat16),
                                    preferred_element_type=jnp.float32)
        sums_ref[:, :] += jnp.dot(oh_bf, xb.astype(jnp.bfloat16),
                                  preferred_element_type=jnp.float32)

        @pl.when(i == nblk - 1)
        def _epilogue():
            lk = (lax.broadcasted_iota(jnp.int32, (K, K), 1)
                  < lax.broadcasted_iota(jnp.int32, (K, K), 0)
                  ).astype(jnp.float32)
            off_ref[:, :] = jnp.dot(lk, counts_ref[:, :],
                                    preferred_element_type=jnp.float32)

    @pl.when(p == 1)
    def _phase1():
        predb = pred_s[pl.ds(i, 1), :]
        poslb = posl_s[pl.ds(i, 1), :]
        oh = (kio == predb).astype(jnp.float32)
        off_t = off_ref[:, :].reshape(1, K)
        offb = jnp.dot(off_t, oh, preferred_element_type=jnp.float32)
        pos = poslb + offb.astype(jnp.int32)
        pos_ref[:, :, :] = pos.reshape(1, 1, BLK)
        xb = x_ref[:, :]
        xp_ref[:, :] = jnp.concatenate([xb, jnp.zeros_like(xb)], axis=1)


def _run_stage_a(x, xt, centers, lt):
    nblk = N // BLK
    return pl.pallas_call(
        functools.partial(_stage_a, nblk=nblk),
        grid=(2, nblk),
        in_specs=[
            pl.BlockSpec((D, BLK), lambda p, i: (0, i)),
            pl.BlockSpec((BLK, D), lambda p, i: (i, 0)),
            pl.BlockSpec((K, D), lambda p, i: (0, 0)),
            pl.BlockSpec((BLK, BLK), lambda p, i: (0, 0)),
        ],
        out_specs=[
            pl.BlockSpec((1, 1, BLK),
                         lambda p, i: (jnp.where(p == 0, 0, i), 0, 0)),
            pl.BlockSpec((K, 1), lambda p, i: (0, 0)),
            pl.BlockSpec((K, D), lambda p, i: (0, 0)),
            pl.BlockSpec((K, 1), lambda p, i: (0, 0)),
            pl.BlockSpec((BLK, 2 * D),
                         lambda p, i: (jnp.where(p == 1, i, 0), 0)),
        ],
        out_shape=[
            jax.ShapeDtypeStruct((nblk, 1, BLK), jnp.int32),
            jax.ShapeDtypeStruct((K, 1), jnp.float32),
            jax.ShapeDtypeStruct((K, D), jnp.float32),
            jax.ShapeDtypeStruct((K, 1), jnp.float32),
            jax.ShapeDtypeStruct((N, 2 * D), jnp.float32),
        ],
        scratch_shapes=[
            pltpu.VMEM((nblk, BLK), jnp.int32),
            pltpu.VMEM((nblk, BLK), jnp.int32),
        ],
        compiler_params=pltpu.CompilerParams(
            dimension_semantics=("arbitrary", "arbitrary"),
        ),
    )(xt, x, centers, lt)


def _scatter_body(x_hbm, pos_hbm, out_hbm, idx_v, rows_v, sem):
    wid = lax.axis_index("s") * 2 + lax.axis_index("c")
    base = wid * PW
    pltpu.sync_copy(pos_hbm.at[pl.ds(base, PW)], idx_v)
    pltpu.sync_copy(x_hbm.at[pl.ds(base, PW)], rows_v)
    pltpu.async_copy(rows_v, out_hbm.at[idx_v], sem).wait()


def _scatter_rows(xp, pos_i):
    mesh = plsc.VectorSubcoreMesh(core_axis_name="c", subcore_axis_name="s")
    fn = pl.kernel(
        _scatter_body,
        out_type=jax.ShapeDtypeStruct((N, 2 * D), jnp.float32),
        mesh=mesh,
        scratch_types=[
            pltpu.VMEM((PW,), jnp.int32),
            pltpu.VMEM((PW, 2 * D), jnp.float32),
            pltpu.SemaphoreType.DMA,
        ],
    )
    return fn(xp, pos_i)


def _gram_body(xs_ref, p_ref, acc):
    b = pl.program_id(0)

    @pl.when(b == 0)
    def _init():
        acc[:, :] = jnp.zeros_like(acc)

    p_ref[:, :] = acc[:, :]
    xsb = xs_ref[:, :D].astype(jnp.bfloat16)
    acc[:, :] += lax.dot_general(xsb, xsb, (((0,), (0,)), ((), ())),
                                 preferred_element_type=jnp.float32)


def _run_gram(xs):
    return pl.pallas_call(
        _gram_body,
        grid=(NGB,),
        in_specs=[pl.BlockSpec((GB, 2 * D), lambda b: (b, 0))],
        out_specs=pl.BlockSpec((D, D), lambda b: (b, 0)),
        out_shape=jax.ShapeDtypeStruct((NGB * D, D), jnp.float32),
        scratch_shapes=[pltpu.VMEM((D, D), jnp.float32)],
        compiler_params=pltpu.CompilerParams(
            dimension_semantics=("arbitrary",),
        ),
    )(xs)


def _stage_d(bep_ref, mlim_ref, xs_ref, p_ref,
             counts_ref, sums_ref, ft_ref, mt_ref, ct_ref, out_ref,
             sedge, prevt):
    k = pl.program_id(0)

    @pl.when(k == 0)
    def _init():
        prevt[:, :] = jnp.zeros_like(prevt)

    xsb = xs_ref[:, :D]
    mlim = mlim_ref[k]
    msk = (lax.broadcasted_iota(jnp.int32, (GB, 1), 0) < mlim
           ).astype(jnp.float32)
    xs_bf = xsb.astype(jnp.bfloat16)
    xm_bf = (xsb * msk).astype(jnp.bfloat16)
    pe = lax.dot_general(xm_bf, xs_bf, (((0,), (0,)), ((), ())),
                         preferred_element_type=jnp.float32)
    tk = p_ref[:, :] + pe
    sedge[pl.ds(k * D, D), :] = tk - prevt[:, :]
    prevt[:, :] = tk

    @pl.when(k == K - 1)
    def _epilogue():
        s_flat = sedge[:, :]
        counts = counts_ref[:, :]
        safe = jnp.maximum(counts, 1.0)
        means = sums_ref[:, :] / safe

        filling = counts / jnp.float32(N)
        loss_fil = jnp.sum((filling - ft_ref[:, :]) ** 2,
                           axis=(0, 1), keepdims=True) / jnp.float32(K)
        loss_means = jnp.sum((means - mt_ref[:, :]) ** 2,
                             axis=(0, 1), keepdims=True) / jnp.float32(K * D)

        m3 = jnp.reshape(jnp.broadcast_to(means[:, None, :], (K, D, D)),
                         (K * D, D))
        rio = lax.broadcasted_iota(jnp.int32, (K * D, D), 0)
        jio = lax.broadcasted_iota(jnp.int32, (K * D, D), 1)
        isel = (rio % D == jio).astype(jnp.float32)
        m4 = jnp.sum(m3 * isel, axis=1, keepdims=True)

        countsb = jnp.reshape(jnp.broadcast_to(counts[:, :, None], (K, D, 1)),
                              (K * D, 1))
        denomb = jnp.maximum(countsb - 1.0, 1.0)
        covs = (s_flat - countsb * (m4 * m3)) / denomb
        loss_covs = jnp.sum((covs - ct_ref[:, :]) ** 2,
                            axis=(0, 1), keepdims=True) / jnp.float32(K * D * D)

        out_ref[:, :] = loss_fil + loss_means + loss_covs


def _run_stage_d(bep, mlim, xs, pgram, counts, sums, ft, mt, ct):
    grid_spec = pltpu.PrefetchScalarGridSpec(
        num_scalar_prefetch=2,
        grid=(K,),
        in_specs=[
            pl.BlockSpec((GB, 2 * D), lambda k, bep, mlim: (bep[k], 0)),
            pl.BlockSpec((D, D), lambda k, bep, mlim: (bep[k], 0)),
            pl.BlockSpec((K, 1), lambda k, bep, mlim: (0, 0)),
            pl.BlockSpec((K, D), lambda k, bep, mlim: (0, 0)),
            pl.BlockSpec((K, 1), lambda k, bep, mlim: (0, 0)),
            pl.BlockSpec((K, D), lambda k, bep, mlim: (0, 0)),
            pl.BlockSpec((K * D, D), lambda k, bep, mlim: (0, 0)),
        ],
        out_specs=pl.BlockSpec((1, 1), lambda k, bep, mlim: (0, 0)),
        scratch_shapes=[
            pltpu.VMEM((K * D, D), jnp.float32),
            pltpu.VMEM((D, D), jnp.float32),
        ],
    )
    return pl.pallas_call(
        _stage_d,
        grid_spec=grid_spec,
        out_shape=jax.ShapeDtypeStruct((1, 1), jnp.float32),
        compiler_params=pltpu.CompilerParams(
            dimension_semantics=("arbitrary",),
        ),
    )(bep, mlim, xs, pgram, counts, sums, ft, mt, ct)


def kernel(x, cluster_centers, filling_target, means_target, covs_target):
    xt = x.T
    jio = lax.broadcasted_iota(jnp.int32, (BLK, BLK), 1)
    iio = lax.broadcasted_iota(jnp.int32, (BLK, BLK), 0)
    lt = (iio < jio).astype(jnp.bfloat16)

    pos3, counts, sums, off, xp = _run_stage_a(x, xt, cluster_centers, lt)

    pos_i = pos3.reshape(N)
    off_i = off.astype(jnp.int32).reshape(K)
    xs = _scatter_rows(xp, pos_i)

    pgram = _run_gram(xs)

    e_i = off_i + counts.astype(jnp.int32).reshape(K)
    bep = jnp.clip(e_i // GB, 0, NGB - 1).astype(jnp.int32)
    mlim = (e_i - bep * GB).astype(jnp.int32)

    ft = filling_target.reshape(K, 1)
    ct = covs_target.reshape(K * D, D)
    out = _run_stage_d(bep, mlim, xs, pgram, counts, sums,
                       ft, means_target, ct)
    return out[0, 0]


# speedup vs baseline: 1.0682x; 1.0324x over previous
import functools

import jax
import jax.numpy as jnp
from jax import lax
from jax.experimental import pallas as pl
from jax.experimental.pallas import tpu as pltpu
from jax.experimental.pallas import tpu_sc as plsc

N, K, D = 16384, 64, 64
BLK = 512
GB = 512
NGB = N // GB
NW = 32
PW = N // NW


def _stage_a(xt_ref, x_ref, c_ref, lt_ref,
             pos_ref, counts_ref, sums_ref, off_ref, xp_ref,
             pred_s, posl_s, *, nblk):
    p = pl.program_id(0)
    i = pl.program_id(1)

    @pl.when((p == 0) & (i == 0))
    def _init():
        counts_ref[:, :] = jnp.zeros_like(counts_ref)
        sums_ref[:, :] = jnp.zeros_like(sums_ref)

    kio = lax.broadcasted_iota(jnp.int32, (K, BLK), 0)

    @pl.when(p == 0)
    def _phase0():
        xt = xt_ref[:, :]
        xb = x_ref[:, :]
        c = c_ref[:, :]

        cn = jnp.sum(c * c, axis=1, keepdims=True)
        xn = jnp.sum(xt * xt, axis=0, keepdims=True)
        d2 = cn - 2.0 * jnp.dot(c, xt, preferred_element_type=jnp.float32) + xn

        dmin = jnp.min(d2, axis=0, keepdims=True)
        pred = jnp.min(jnp.where(d2 <= dmin, kio, K), axis=0, keepdims=True)
        oh = (kio == pred).astype(jnp.float32)
        oh_bf = oh.astype(jnp.bfloat16)

        cum = jnp.dot(oh_bf, lt_ref[:, :], preferred_element_type=jnp.float32)
        carry = counts_ref[:, :]
        posl = jnp.sum(oh * (cum + carry), axis=0, keepdims=True)

        pred_s[pl.ds(i, 1), :] = pred
        posl_s[pl.ds(i, 1), :] = posl.astype(jnp.int32)

        counts_ref[:, :] = carry + jnp.sum(oh, axis=1, keepdims=True)
        sums_ref[:, :] += jnp.dot(oh_bf, xb.astype(jnp.bfloat16),
                                  preferred_element_type=jnp.float32)

        @pl.when(i == nblk - 1)
        def _epilogue():
            lk = (lax.broadcasted_iota(jnp.int32, (K, K), 1)
                  < lax.broadcasted_iota(jnp.int32, (K, K), 0)
                  ).astype(jnp.float32)
            off_ref[:, :] = jnp.dot(lk, counts_ref[:, :],
                                    preferred_element_type=jnp.float32)

    @pl.when(p == 1)
    def _phase1():
        predb = pred_s[pl.ds(i, 1), :]
        poslb = posl_s[pl.ds(i, 1), :]
        oh = (kio == predb).astype(jnp.float32)
        offb = jnp.sum(oh * off_ref[:, :], axis=0, keepdims=True)
        pos = poslb + offb.astype(jnp.int32)
        pos_ref[:, :, :] = pos.reshape(1, 1, BLK)
        xb = x_ref[:, :]
        xp_ref[:, :] = jnp.concatenate([xb, jnp.zeros_like(xb)], axis=1)


def _run_stage_a(x, xt, centers, lt):
    nblk = N // BLK
    return pl.pallas_call(
        functools.partial(_stage_a, nblk=nblk),
        grid=(2, nblk),
        in_specs=[
            pl.BlockSpec((D, BLK), lambda p, i: (0, i)),
            pl.BlockSpec((BLK, D), lambda p, i: (i, 0)),
            pl.BlockSpec((K, D), lambda p, i: (0, 0)),
            pl.BlockSpec((BLK, BLK), lambda p, i: (0, 0)),
        ],
        out_specs=[
            pl.BlockSpec((1, 1, BLK),
                         lambda p, i: (jnp.where(p == 0, 0, i), 0, 0)),
            pl.BlockSpec((K, 1), lambda p, i: (0, 0)),
            pl.BlockSpec((K, D), lambda p, i: (0, 0)),
            pl.BlockSpec((K, 1), lambda p, i: (0, 0)),
            pl.BlockSpec((BLK, 2 * D),
                         lambda p, i: (jnp.where(p == 1, i, 0), 0)),
        ],
        out_shape=[
            jax.ShapeDtypeStruct((nblk, 1, BLK), jnp.int32),
            jax.ShapeDtypeStruct((K, 1), jnp.float32),
            jax.ShapeDtypeStruct((K, D), jnp.float32),
            jax.ShapeDtypeStruct((K, 1), jnp.float32),
            jax.ShapeDtypeStruct((N, 2 * D), jnp.float32),
        ],
        scratch_shapes=[
            pltpu.VMEM((nblk, BLK), jnp.int32),
            pltpu.VMEM((nblk, BLK), jnp.int32),
        ],
        compiler_params=pltpu.CompilerParams(
            dimension_semantics=("arbitrary", "arbitrary"),
        ),
    )(xt, x, centers, lt)


def _scatter_body(x_hbm, pos_hbm, out_hbm, idx_v, rows_v, sem):
    wid = lax.axis_index("s") * 2 + lax.axis_index("c")
    base = wid * PW
    pltpu.sync_copy(pos_hbm.at[pl.ds(base, PW)], idx_v)
    pltpu.sync_copy(x_hbm.at[pl.ds(base, PW)], rows_v)
    pltpu.async_copy(rows_v, out_hbm.at[idx_v], sem).wait()


def _scatter_rows(xp, pos_i):
    mesh = plsc.VectorSubcoreMesh(core_axis_name="c", subcore_axis_name="s")
    fn = pl.kernel(
        _scatter_body,
        out_type=jax.ShapeDtypeStruct((N, 2 * D), jnp.float32),
        mesh=mesh,
        scratch_types=[
            pltpu.VMEM((PW,), jnp.int32),
            pltpu.VMEM((PW, 2 * D), jnp.float32),
            pltpu.SemaphoreType.DMA,
        ],
    )
    return fn(xp, pos_i)


def _gram_body(xs_ref, p_ref, acc):
    b = pl.program_id(0)

    @pl.when(b == 0)
    def _init():
        acc[:, :] = jnp.zeros_like(acc)

    p_ref[:, :] = acc[:, :]
    xsb = xs_ref[:, :D].astype(jnp.bfloat16)
    acc[:, :] += lax.dot_general(xsb, xsb, (((0,), (0,)), ((), ())),
                                 preferred_element_type=jnp.float32)


def _run_gram(xs):
    return pl.pallas_call(
        _gram_body,
        grid=(NGB,),
        in_specs=[pl.BlockSpec((GB, 2 * D), lambda b: (b, 0))],
        out_specs=pl.BlockSpec((D, D), lambda b: (b, 0)),
        out_shape=jax.ShapeDtypeStruct((NGB * D, D), jnp.float32),
        scratch_shapes=[pltpu.VMEM((D, D), jnp.float32)],
        compiler_params=pltpu.CompilerParams(
            dimension_semantics=("arbitrary",),
        ),
    )(xs)


def _stage_d(bep_ref, mlim_ref, xs_ref, p_ref,
             counts_ref, sums_ref, ft_ref, mt_ref, ct_ref, out_ref,
             sedge, prevt):
    k = pl.program_id(0)

    @pl.when(k == 0)
    def _init():
        prevt[:, :] = jnp.zeros_like(prevt)

    xsb = xs_ref[:, :D]
    mlim = mlim_ref[k]
    msk = (lax.broadcasted_iota(jnp.int32, (GB, 1), 0) < mlim
           ).astype(jnp.float32)
    xs_bf = xsb.astype(jnp.bfloat16)
    xm_bf = (xsb * msk).astype(jnp.bfloat16)
    pe = lax.dot_general(xm_bf, xs_bf, (((0,), (0,)), ((), ())),
                         preferred_element_type=jnp.float32)
    tk = p_ref[:, :] + pe
    sedge[pl.ds(k * D, D), :] = tk - prevt[:, :]
    prevt[:, :] = tk

    @pl.when(k == K - 1)
    def _epilogue():
        s_flat = sedge[:, :]
        counts = counts_ref[:, :]
        safe = jnp.maximum(counts, 1.0)
        means = sums_ref[:, :] / safe

        filling = counts / jnp.float32(N)
        loss_fil = jnp.sum((filling - ft_ref[:, :]) ** 2,
                           axis=(0, 1), keepdims=True) / jnp.float32(K)
        loss_means = jnp.sum((means - mt_ref[:, :]) ** 2,
                             axis=(0, 1), keepdims=True) / jnp.float32(K * D)

        m3 = jnp.reshape(jnp.broadcast_to(means[:, None, :], (K, D, D)),
                         (K * D, D))
        rio = lax.broadcasted_iota(jnp.int32, (K * D, D), 0)
        jio = lax.broadcasted_iota(jnp.int32, (K * D, D), 1)
        isel = (rio % D == jio).astype(jnp.float32)
        m4 = jnp.sum(m3 * isel, axis=1, keepdims=True)

        countsb = jnp.reshape(jnp.broadcast_to(counts[:, :, None], (K, D, 1)),
                              (K * D, 1))
        denomb = jnp.maximum(countsb - 1.0, 1.0)
        covs = (s_flat - countsb * (m4 * m3)) / denomb
        loss_covs = jnp.sum((covs - ct_ref[:, :]) ** 2,
                            axis=(0, 1), keepdims=True) / jnp.float32(K * D * D)

        out_ref[:, :] = loss_fil + loss_means + loss_covs


def _run_stage_d(bep, mlim, xs, pgram, counts, sums, ft, mt, ct):
    grid_spec = pltpu.PrefetchScalarGridSpec(
        num_scalar_prefetch=2,
        grid=(K,),
        in_specs=[
            pl.BlockSpec((GB, 2 * D), lambda k, bep, mlim: (bep[k], 0)),
            pl.BlockSpec((D, D), lambda k, bep, mlim: (bep[k], 0)),
            pl.BlockSpec((K, 1), lambda k, bep, mlim: (0, 0)),
            pl.BlockSpec((K, D), lambda k, bep, mlim: (0, 0)),
            pl.BlockSpec((K, 1), lambda k, bep, mlim: (0, 0)),
            pl.BlockSpec((K, D), lambda k, bep, mlim: (0, 0)),
            pl.BlockSpec((K * D, D), lambda k, bep, mlim: (0, 0)),
        ],
        out_specs=pl.BlockSpec((1, 1), lambda k, bep, mlim: (0, 0)),
        scratch_shapes=[
            pltpu.VMEM((K * D, D), jnp.float32),
            pltpu.VMEM((D, D), jnp.float32),
        ],
    )
    return pl.pallas_call(
        _stage_d,
        grid_spec=grid_spec,
        out_shape=jax.ShapeDtypeStruct((1, 1), jnp.float32),
        compiler_params=pltpu.CompilerParams(
            dimension_semantics=("arbitrary",),
        ),
    )(bep, mlim, xs, pgram, counts, sums, ft, mt, ct)


def kernel(x, cluster_centers, filling_target, means_target, covs_target):
    xt = x.T
    jio = lax.broadcasted_iota(jnp.int32, (BLK, BLK), 1)
    iio = lax.broadcasted_iota(jnp.int32, (BLK, BLK), 0)
    lt = (iio < jio).astype(jnp.bfloat16)

    pos3, counts, sums, off, xp = _run_stage_a(x, xt, cluster_centers, lt)

    pos_i = pos3.reshape(N)
    off_i = off.astype(jnp.int32).reshape(K)
    xs = _scatter_rows(xp, pos_i)

    pgram = _run_gram(xs)

    e_i = off_i + counts.astype(jnp.int32).reshape(K)
    bep = jnp.clip(e_i // GB, 0, NGB - 1).astype(jnp.int32)
    mlim = (e_i - bep * GB).astype(jnp.int32)

    ft = filling_target.reshape(K, 1)
    ct = covs_target.reshape(K * D, D)
    out = _run_stage_d(bep, mlim, xs, pgram, counts, sums,
                       ft, means_target, ct)
    return out[0, 0]
